# Initial kernel scaffold; baseline (speedup 1.0000x reference)
import functools

import jax
import jax.numpy as jnp
from jax import lax
from jax.experimental import pallas as pl
from jax.experimental.pallas import tpu as pltpu
from jax.experimental.pallas import tpu_sc as plsc

N = 10000
H = 256
HH = 128
PAD_N = 10240
TRASH = 10200
E = 160000
E_PAD = 163840
ECH = 80
EL = 100000
EL_PAD = 102400
LCH = 25
CW = 16

_MESH = plsc.VectorSubcoreMesh(
    core_axis_name="c", subcore_axis_name="s", num_cores=2, num_subcores=16)

_ZERO16 = jnp.zeros((16,), jnp.float32)
_ONE16 = jnp.ones((16,), jnp.float32)


def _fill_rows(ref, nrows, ncols, value16):
    def row(r, _):
        for j in range(ncols // 16):
            ref[r, pl.ds(16 * j, 16)] = value16
        return 0
    lax.fori_loop(0, nrows, row, 0)


def _agg_body(with_counts, *refs):
    if with_counts:
        (y_hbm, gidx_hbm, sidx_hbm, out_hbm, cnt_hbm,
         gidx_v, sidx_v, rows_v, ones_v, zc_v, acc_sh, cnt_sh, sem) = refs
    else:
        (y_hbm, gidx_hbm, sidx_hbm, out_hbm,
         gidx_v, sidx_v, rows_v, acc_sh, sem) = refs
    c = lax.axis_index("c")
    s = lax.axis_index("s")

    _fill_rows(rows_v, 128, HH, _ZERO16)
    for k in range(5):
        pltpu.sync_copy(rows_v, acc_sh.at[pl.ds(s * 640 + k * 128, 128)])
    if with_counts:
        _fill_rows(ones_v, 128, CW, _ONE16)
        _fill_rows(zc_v, 128, CW, _ZERO16)
        @pl.when(c == 0)
        def _():
            for k in range(5):
                pltpu.sync_copy(zc_v, cnt_sh.at[pl.ds(s * 640 + k * 128, 128)])
    plsc.subcore_barrier()

    pltpu.sync_copy(gidx_hbm.at[s], gidx_v)
    pltpu.sync_copy(sidx_hbm.at[s], sidx_v)

    def chunk(j, _):
        pltpu.async_copy(y_hbm.at[c].at[gidx_v.at[j]], rows_v, sem).wait()
        pltpu.sync_copy(rows_v, acc_sh.at[sidx_v.at[j]], add=True)
        if with_counts:
            @pl.when(c == 0)
            def _():
                pltpu.sync_copy(ones_v, cnt_sh.at[sidx_v.at[j]], add=True)
        return 0
    lax.fori_loop(0, ECH, chunk, 0)

    plsc.subcore_barrier()
    pltpu.sync_copy(acc_sh.at[pl.ds(s * 640, 640)],
                    out_hbm.at[c].at[pl.ds(s * 640, 640)])
    if with_counts:
        @pl.when(c == 0)
        def _():
            pltpu.sync_copy(cnt_sh.at[pl.ds(s * 640, 640)],
                            cnt_hbm.at[pl.ds(s * 640, 640)])


def _make_agg(with_counts):
    out_type = [jax.ShapeDtypeStruct((2, PAD_N, HH), jnp.float32)]
    scratch = [
        pltpu.VMEM((ECH, 128), jnp.int32),
        pltpu.VMEM((ECH, 128), jnp.int32),
        pltpu.VMEM((128, HH), jnp.float32),
    ]
    if with_counts:
        out_type.append(jax.ShapeDtypeStruct((PAD_N, CW), jnp.float32))
        scratch += [
            pltpu.VMEM((128, CW), jnp.float32),
            pltpu.VMEM((128, CW), jnp.float32),
        ]
    scratch.append(plsc.MemoryRef((PAD_N, HH), jnp.float32, pltpu.VMEM_SHARED))
    if with_counts:
        scratch.append(
            plsc.MemoryRef((PAD_N, CW), jnp.float32, pltpu.VMEM_SHARED))
    scratch.append(pltpu.SemaphoreType.DMA)
    return pl.kernel(
        functools.partial(_agg_body, with_counts),
        out_type=tuple(out_type) if with_counts else out_type[0],
        mesh=_MESH,
        scratch_types=scratch,
    )


def _dec_body(xu_hbm, xm_hbm, i0_hbm, i1_hbm, out_hbm,
              i0_v, i1_v, fu_v, fm_v, out_v, sem0, sem1):
    c = lax.axis_index("c")
    s = lax.axis_index("s")
    w = s * 2 + c
    pltpu.sync_copy(i0_hbm.at[w], i0_v)
    pltpu.sync_copy(i1_hbm.at[w], i1_v)

    def chunk(j, _):
        d0 = pltpu.async_copy(xu_hbm.at[i0_v.at[j]], fu_v, sem0)
        d1 = pltpu.async_copy(xm_hbm.at[i1_v.at[j]], fm_v, sem1)
        d0.wait()
        d1.wait()

        def edge(e, _):
            acc = fu_v[e, pl.ds(0, 16)] * fm_v[e, pl.ds(0, 16)]
            for k in range(1, 16):
                acc = acc + fu_v[e, pl.ds(16 * k, 16)] * fm_v[e, pl.ds(16 * k, 16)]
            out_v[j, e] = jnp.sum(acc)
            return 0
        lax.fori_loop(0, 128, edge, 0)
        return 0
    lax.fori_loop(0, LCH, chunk, 0)
    pltpu.sync_copy(out_v, out_hbm.at[w])


_decoder = pl.kernel(
    _dec_body,
    out_type=jax.ShapeDtypeStruct((32, LCH, 128), jnp.float32),
    mesh=_MESH,
    scratch_types=[
        pltpu.VMEM((LCH, 128), jnp.int32),
        pltpu.VMEM((LCH, 128), jnp.int32),
        pltpu.VMEM((128, H), jnp.float32),
        pltpu.VMEM((128, H), jnp.float32),
        pltpu.VMEM((LCH, 128), jnp.float32),
        pltpu.SemaphoreType.DMA,
        pltpu.SemaphoreType.DMA,
    ],
)


RB = 1000
GRID = N // RB


def _t0_body(mx_ref, me_ref, xu_ref, linW_ref, linb_ref,
             Wl1um_ref, Wl1mu_ref, xm_ref, yu1_ref, ym1_ref):
    xm = (jnp.dot(mx_ref[...], linW_ref[...], preferred_element_type=jnp.float32)
          + linb_ref[...] + me_ref[...])
    xm_ref[...] = xm
    yu1 = jnp.dot(xu_ref[...], Wl1um_ref[...], preferred_element_type=jnp.float32)
    ym1 = jnp.dot(xm, Wl1mu_ref[...], preferred_element_type=jnp.float32)
    yu1_ref[0] = yu1[:, :HH]
    yu1_ref[1] = yu1[:, HH:]
    ym1_ref[0] = ym1[:, :HH]
    ym1_ref[1] = ym1[:, HH:]


def _row_spec(width):
    return pl.BlockSpec((RB, width), lambda i: (i, 0))


def _split_spec():
    return pl.BlockSpec((2, RB, HH), lambda i: (0, i, 0))


def _full_spec(shape):
    return pl.BlockSpec(shape, lambda i: tuple(0 for _ in shape))


def _t0(movie_x, movie_emb, user_emb, lin_W, lin_b, Wl1_um, Wl1_mu):
    return pl.pallas_call(
        _t0_body,
        grid=(GRID,),
        in_specs=[
            _row_spec(20), _row_spec(H), _row_spec(H),
            _full_spec((20, H)), _full_spec((1, H)),
            _full_spec((H, H)), _full_spec((H, H)),
        ],
        out_specs=[_row_spec(H), _split_spec(), _split_spec()],
        out_shape=[
            jax.ShapeDtypeStruct((N, H), jnp.float32),
            jax.ShapeDtypeStruct((2, N, HH), jnp.float32),
            jax.ShapeDtypeStruct((2, N, HH), jnp.float32),
        ],
    )(movie_x, movie_emb, user_emb, lin_W, lin_b.reshape(1, H), Wl1_um, Wl1_mu)


def _t1_body(aggm_ref, aggu_ref, cntm_ref, cntu_ref, xm_ref, xu_ref,
             bl1um_ref, Wr1um_ref, bl1mu_ref, Wr1mu_ref,
             Wl2um_ref, Wl2mu_ref,
             xm1_ref, xu1_ref, yu2_ref, ym2_ref):
    rm = 1.0 / jnp.maximum(cntm_ref[...][:, :1], 1.0)
    ru = 1.0 / jnp.maximum(cntu_ref[...][:, :1], 1.0)
    aggm = jnp.concatenate([aggm_ref[0], aggm_ref[1]], axis=-1) * rm
    aggu = jnp.concatenate([aggu_ref[0], aggu_ref[1]], axis=-1) * ru
    xm1 = jax.nn.relu(
        aggm + bl1um_ref[...]
        + jnp.dot(xm_ref[...], Wr1um_ref[...], preferred_element_type=jnp.float32))
    xu1 = jax.nn.relu(
        aggu + bl1mu_ref[...]
        + jnp.dot(xu_ref[...], Wr1mu_ref[...], preferred_element_type=jnp.float32))
    xm1_ref[...] = xm1
    xu1_ref[...] = xu1
    yu2 = jnp.dot(xu1, Wl2um_ref[...], preferred_element_type=jnp.float32)
    ym2 = jnp.dot(xm1, Wl2mu_ref[...], preferred_element_type=jnp.float32)
    yu2_ref[0] = yu2[:, :HH]
    yu2_ref[1] = yu2[:, HH:]
    ym2_ref[0] = ym2[:, :HH]
    ym2_ref[1] = ym2[:, HH:]


def _t1(aggm, aggu, cntm, cntu, xm, xu, bl1_um, Wr1_um, bl1_mu, Wr1_mu,
        Wl2_um, Wl2_mu):
    return pl.pallas_call(
        _t1_body,
        grid=(GRID,),
        in_specs=[
            _split_spec(), _split_spec(),
            _row_spec(CW), _row_spec(CW),
            _row_spec(H), _row_spec(H),
            _full_spec((1, H)), _full_spec((H, H)),
            _full_spec((1, H)), _full_spec((H, H)),
            _full_spec((H, H)), _full_spec((H, H)),
        ],
        out_specs=[_row_spec(H), _row_spec(H), _split_spec(), _split_spec()],
        out_shape=[
            jax.ShapeDtypeStruct((N, H), jnp.float32),
            jax.ShapeDtypeStruct((N, H), jnp.float32),
            jax.ShapeDtypeStruct((2, N, HH), jnp.float32),
            jax.ShapeDtypeStruct((2, N, HH), jnp.float32),
        ],
    )(aggm, aggu, cntm, cntu, xm, xu, bl1_um.reshape(1, H), Wr1_um,
      bl1_mu.reshape(1, H), Wr1_mu, Wl2_um, Wl2_mu)


def _t2_body(aggm_ref, aggu_ref, cntm_ref, cntu_ref, xm1_ref, xu1_ref,
             bl2um_ref, Wr2um_ref, bl2mu_ref, Wr2mu_ref,
             xm2_ref, xu2_ref):
    rm = 1.0 / jnp.maximum(cntm_ref[...][:, :1], 1.0)
    ru = 1.0 / jnp.maximum(cntu_ref[...][:, :1], 1.0)
    aggm = jnp.concatenate([aggm_ref[0], aggm_ref[1]], axis=-1) * rm
    aggu = jnp.concatenate([aggu_ref[0], aggu_ref[1]], axis=-1) * ru
    xm2_ref[...] = (
        aggm + bl2um_ref[...]
        + jnp.dot(xm1_ref[...], Wr2um_ref[...], preferred_element_type=jnp.float32))
    xu2_ref[...] = (
        aggu + bl2mu_ref[...]
        + jnp.dot(xu1_ref[...], Wr2mu_ref[...], preferred_element_type=jnp.float32))


def _t2(aggm, aggu, cntm, cntu, xm1, xu1, bl2_um, Wr2_um, bl2_mu, Wr2_mu):
    return pl.pallas_call(
        _t2_body,
        grid=(GRID,),
        in_specs=[
            _split_spec(), _split_spec(),
            _row_spec(CW), _row_spec(CW),
            _row_spec(H), _row_spec(H),
            _full_spec((1, H)), _full_spec((H, H)),
            _full_spec((1, H)), _full_spec((H, H)),
        ],
        out_specs=[_row_spec(H), _row_spec(H)],
        out_shape=[
            jax.ShapeDtypeStruct((N, H), jnp.float32),
            jax.ShapeDtypeStruct((N, H), jnp.float32),
        ],
    )(aggm, aggu, cntm, cntu, xm1, xu1, bl2_um.reshape(1, H), Wr2_um,
      bl2_mu.reshape(1, H), Wr2_mu)


def kernel(user_node_id, movie_node_id, movie_x, edge_index, edge_label_index,
           user_emb, movie_emb, lin_W, lin_b,
           Wl1_um, bl1_um, Wr1_um, Wl1_mu, bl1_mu, Wr1_mu,
           Wl2_um, bl2_um, Wr2_um, Wl2_mu, bl2_mu, Wr2_mu):
    xu = user_emb
    src = edge_index[0]
    dst = edge_index[1]

    pad_e = E_PAD - E
    g_src = jnp.concatenate([src, jnp.zeros((pad_e,), jnp.int32)]).reshape(16, ECH, 128)
    s_src = jnp.concatenate([src, jnp.full((pad_e,), TRASH, jnp.int32)]).reshape(16, ECH, 128)
    g_dst = jnp.concatenate([dst, jnp.zeros((pad_e,), jnp.int32)]).reshape(16, ECH, 128)
    s_dst = jnp.concatenate([dst, jnp.full((pad_e,), TRASH, jnp.int32)]).reshape(16, ECH, 128)

    pad_l = EL_PAD - EL
    g_l0 = jnp.concatenate([edge_label_index[0], jnp.zeros((pad_l,), jnp.int32)]).reshape(32, LCH, 128)
    g_l1 = jnp.concatenate([edge_label_index[1], jnp.zeros((pad_l,), jnp.int32)]).reshape(32, LCH, 128)

    xm, yu1, ym1 = _t0(movie_x, movie_emb, xu, lin_W, lin_b, Wl1_um, Wl1_mu)

    agg_c = _make_agg(True)
    agg_n = _make_agg(False)

    aggm1, cntm = agg_c(yu1, g_src, s_dst)
    aggu1, cntu = agg_c(ym1, g_dst, s_src)

    xm1, xu1, yu2, ym2 = _t1(aggm1, aggu1, cntm, cntu, xm, xu,
                             bl1_um, Wr1_um, bl1_mu, Wr1_mu, Wl2_um, Wl2_mu)

    aggm2 = agg_n(yu2, g_src, s_dst)
    aggu2 = agg_n(ym2, g_dst, s_src)

    xm2, xu2 = _t2(aggm2, aggu2, cntm, cntu, xm1, xu1,
                   bl2_um, Wr2_um, bl2_mu, Wr2_mu)

    pred = _decoder(xu2, xm2, g_l0, g_l1)
    return pred.reshape(EL_PAD)[:EL]


# speedup vs baseline: 2.7301x; 2.7301x over previous
import functools

import jax
import jax.numpy as jnp
from jax import lax
from jax.experimental import pallas as pl
from jax.experimental.pallas import tpu as pltpu
from jax.experimental.pallas import tpu_sc as plsc

N = 10000
H = 256
HH = 128
PAD_N = 10240
TRASH = 10200
E = 160000
E_PAD = 163840
ECH = 80
EL = 100000
EL_PAD = 102400
LCH = 25
CW = 128

@functools.lru_cache(maxsize=None)
def _mesh():
    return plsc.VectorSubcoreMesh(
        core_axis_name="c", subcore_axis_name="s", num_cores=2, num_subcores=16)


_SC_PARAMS = pltpu.CompilerParams(needs_layout_passes=False)

def _fill_rows(ref, nrows, ncols, value):
    value16 = jnp.full((16,), value, jnp.float32)
    def row(r, _):
        for j in range(ncols // 16):
            ref[r, pl.ds(16 * j, 16)] = value16
        return 0
    lax.fori_loop(0, nrows, row, 0)


NBUF = 4
CHUNK = 64
CPT = E_PAD // (16 * CHUNK)
HCPT = CPT // 4


def _agg_body(y_hbm, gidx_hbm, sidx_hbm, out_hbm,
              gidx_v, sidx_v, rows_v, acc_sh, *sems):
    gsem = sems[:NBUF]
    ssem = sems[NBUF:]
    c = lax.axis_index("c")
    s = lax.axis_index("s")

    _fill_rows(rows_v.at[0], CHUNK, HH, 0.0)
    for k in range(10):
        pltpu.sync_copy(rows_v.at[0], acc_sh.at[pl.ds(s * 640 + k * CHUNK, CHUNK)])
    plsc.subcore_barrier()

    def fire_gather(j, b):
        pltpu.async_copy(y_hbm.at[c].at[gidx_v.at[j]], rows_v.at[b], gsem[b])

    def wait_gather(b):
        pltpu.make_async_copy(
            y_hbm.at[c].at[gidx_v.at[0]], rows_v.at[b], gsem[b]).wait()

    def wait_scatter(b):
        pltpu.make_async_copy(
            rows_v.at[b], acc_sh.at[sidx_v.at[0]], ssem[b]).wait()

    for h in range(4):
        pltpu.sync_copy(gidx_hbm.at[s].at[pl.ds(h * HCPT, HCPT)], gidx_v)
        pltpu.sync_copy(sidx_hbm.at[s].at[pl.ds(h * HCPT, HCPT)], sidx_v)
        for b in range(NBUF):
            fire_gather(b, b)

        def group(g, _):
            base = g * NBUF
            for b in range(NBUF):
                wait_gather(b)
                pltpu.async_copy(
                    rows_v.at[b], acc_sh.at[sidx_v.at[base + b]], ssem[b],
                    add=True)
            nxt = base + NBUF

            @pl.when(nxt < HCPT)
            def _():
                for b in range(NBUF):
                    wait_scatter(b)
                    fire_gather(nxt + b, b)
            return 0
        lax.fori_loop(0, HCPT // NBUF, group, 0)
        for b in range(NBUF):
            wait_scatter(b)

    plsc.subcore_barrier()
    pltpu.sync_copy(acc_sh.at[pl.ds(s * 640, 640)],
                    out_hbm.at[c].at[pl.ds(s * 640, 640)])


@functools.lru_cache(maxsize=None)
def _make_agg():
    return pl.kernel(
        _agg_body,
        out_type=jax.ShapeDtypeStruct((2, PAD_N, HH), jnp.float32),
        mesh=_mesh(),
        compiler_params=_SC_PARAMS,
        scratch_types=[
            pltpu.VMEM((HCPT, CHUNK), jnp.int32),
            pltpu.VMEM((HCPT, CHUNK), jnp.int32),
            pltpu.VMEM((NBUF, CHUNK, HH), jnp.float32),
            pltpu.VMEM_SHARED((PAD_N, HH), jnp.float32),
        ] + [pltpu.SemaphoreType.DMA] * (2 * NBUF),
    )


def _hist_body(src_hbm, dst_hbm, cnt_hbm,
               idx_v, ones_v, cnt_sh, sem):
    c = lax.axis_index("c")
    s = lax.axis_index("s")
    _fill_rows(ones_v, 128, CW, 0.0)
    for k in range(5):
        pltpu.sync_copy(ones_v, cnt_sh.at[pl.ds(s * 640 + k * 128, 128)])
    _fill_rows(ones_v, 128, CW, 1.0)
    plsc.subcore_barrier()

    def do(idx_hbm):
        pltpu.sync_copy(idx_hbm.at[s], idx_v)

        def chunk(j, _):
            pltpu.sync_copy(ones_v, cnt_sh.at[idx_v.at[j]], add=True)
            return 0
        lax.fori_loop(0, ECH, chunk, 0)

    @pl.when(c == 0)
    def _():
        do(src_hbm)

    @pl.when(c == 1)
    def _():
        do(dst_hbm)

    plsc.subcore_barrier()
    pltpu.sync_copy(cnt_sh.at[pl.ds(s * 640, 640)],
                    cnt_hbm.at[c].at[pl.ds(s * 640, 640)])


@functools.lru_cache(maxsize=None)
def _make_hist():
    return pl.kernel(
        _hist_body,
        out_type=jax.ShapeDtypeStruct((2, PAD_N, CW), jnp.float32),
        mesh=_mesh(),
        compiler_params=_SC_PARAMS,
        scratch_types=[
            pltpu.VMEM((ECH, 128), jnp.int32),
            pltpu.VMEM((128, CW), jnp.float32),
            pltpu.VMEM_SHARED((PAD_N, CW), jnp.float32),
            pltpu.SemaphoreType.DMA,
        ],
    )


DCH = 64
DPT = EL_PAD // (32 * DCH)


def _dec_body(xu_hbm, xm_hbm, i0_hbm, i1_hbm, out_hbm,
              i0_v, i1_v, fu_v, fm_v, out_v, *sems):
    usem = sems[:2]
    msem = sems[2:]
    c = lax.axis_index("c")
    s = lax.axis_index("s")
    w = s * 2 + c
    pltpu.sync_copy(i0_hbm.at[w], i0_v)
    pltpu.sync_copy(i1_hbm.at[w], i1_v)
    lane0 = lax.iota(jnp.int32, 16) == 0

    def fire(j, b):
        pltpu.async_copy(xu_hbm.at[i0_v.at[j]], fu_v.at[b], usem[b])
        pltpu.async_copy(xm_hbm.at[i1_v.at[j]], fm_v.at[b], msem[b])

    def wait(b):
        pltpu.make_async_copy(xu_hbm.at[i0_v.at[0]], fu_v.at[b], usem[b]).wait()
        pltpu.make_async_copy(xm_hbm.at[i1_v.at[0]], fm_v.at[b], msem[b]).wait()

    def compute(j, b):
        def edge(e, _):
            acc = fu_v[b, e, pl.ds(0, 16)] * fm_v[b, e, pl.ds(0, 16)]
            for k in range(1, 16):
                acc = acc + (fu_v[b, e, pl.ds(16 * k, 16)]
                             * fm_v[b, e, pl.ds(16 * k, 16)])
            val = jnp.sum(acc)
            pos = j * DCH + e
            plsc.store_scatter(out_v, [jnp.full((16,), pos, jnp.int32)],
                               jnp.full((16,), val, jnp.float32), mask=lane0)
            return 0
        lax.fori_loop(0, DCH, edge, 0)

    fire(0, 0)

    def group(g, _):
        j0 = g * 2
        fire(j0 + 1, 1)
        wait(0)
        compute(j0, 0)

        @pl.when(j0 + 2 < DPT)
        def _():
            fire(j0 + 2, 0)
        wait(1)
        compute(j0 + 1, 1)
        return 0
    lax.fori_loop(0, DPT // 2, group, 0)
    pltpu.sync_copy(out_v, out_hbm.at[w])


@functools.lru_cache(maxsize=None)
def _make_decoder():
    return pl.kernel(
        _dec_body,
        out_type=jax.ShapeDtypeStruct((32, DPT * DCH), jnp.float32),
        mesh=_mesh(),
        compiler_params=_SC_PARAMS,
        scratch_types=[
            pltpu.VMEM((DPT, DCH), jnp.int32),
            pltpu.VMEM((DPT, DCH), jnp.int32),
            pltpu.VMEM((2, DCH, H), jnp.float32),
            pltpu.VMEM((2, DCH, H), jnp.float32),
            pltpu.VMEM((DPT * DCH,), jnp.float32),
        ] + [pltpu.SemaphoreType.DMA] * 4,
    )


RB = 1000
GRID = N // RB


def _t0_body(mx_ref, me_ref, xu_ref, linW_ref, linb_ref,
             Wl1um_ref, Wl1mu_ref, xm_ref, yu1_ref, ym1_ref):
    xm = (jnp.dot(mx_ref[...], linW_ref[...], preferred_element_type=jnp.float32)
          + linb_ref[...] + me_ref[...])
    xm_ref[...] = xm
    yu1 = jnp.dot(xu_ref[...], Wl1um_ref[...], preferred_element_type=jnp.float32)
    ym1 = jnp.dot(xm, Wl1mu_ref[...], preferred_element_type=jnp.float32)
    yu1_ref[0] = yu1[:, :HH]
    yu1_ref[1] = yu1[:, HH:]
    ym1_ref[0] = ym1[:, :HH]
    ym1_ref[1] = ym1[:, HH:]


def _row_spec(width):
    return pl.BlockSpec((RB, width), lambda i: (i, 0))


def _split_spec():
    return pl.BlockSpec((2, RB, HH), lambda i: (0, i, 0))


def _full_spec(shape):
    return pl.BlockSpec(shape, lambda i: tuple(0 for _ in shape))


def _t0(movie_x, movie_emb, user_emb, lin_W, lin_b, Wl1_um, Wl1_mu):
    return pl.pallas_call(
        _t0_body,
        grid=(GRID,),
        in_specs=[
            _row_spec(20), _row_spec(H), _row_spec(H),
            _full_spec((20, H)), _full_spec((1, H)),
            _full_spec((H, H)), _full_spec((H, H)),
        ],
        out_specs=[_row_spec(H), _split_spec(), _split_spec()],
        out_shape=[
            jax.ShapeDtypeStruct((N, H), jnp.float32),
            jax.ShapeDtypeStruct((2, N, HH), jnp.float32),
            jax.ShapeDtypeStruct((2, N, HH), jnp.float32),
        ],
    )(movie_x, movie_emb, user_emb, lin_W, lin_b.reshape(1, H), Wl1_um, Wl1_mu)


def _t1_body(aggm_ref, aggu_ref, cntm_ref, cntu_ref, xm_ref, xu_ref,
             bl1um_ref, Wr1um_ref, bl1mu_ref, Wr1mu_ref,
             Wl2um_ref, Wl2mu_ref,
             xm1_ref, xu1_ref, yu2_ref, ym2_ref):
    rm = 1.0 / jnp.maximum(cntm_ref[...][:, :1], 1.0)
    ru = 1.0 / jnp.maximum(cntu_ref[...][:, :1], 1.0)
    aggm = jnp.concatenate([aggm_ref[0], aggm_ref[1]], axis=-1) * rm
    aggu = jnp.concatenate([aggu_ref[0], aggu_ref[1]], axis=-1) * ru
    xm1 = jax.nn.relu(
        aggm + bl1um_ref[...]
        + jnp.dot(xm_ref[...], Wr1um_ref[...], preferred_element_type=jnp.float32))
    xu1 = jax.nn.relu(
        aggu + bl1mu_ref[...]
        + jnp.dot(xu_ref[...], Wr1mu_ref[...], preferred_element_type=jnp.float32))
    xm1_ref[...] = xm1
    xu1_ref[...] = xu1
    yu2 = jnp.dot(xu1, Wl2um_ref[...], preferred_element_type=jnp.float32)
    ym2 = jnp.dot(xm1, Wl2mu_ref[...], preferred_element_type=jnp.float32)
    yu2_ref[0] = yu2[:, :HH]
    yu2_ref[1] = yu2[:, HH:]
    ym2_ref[0] = ym2[:, :HH]
    ym2_ref[1] = ym2[:, HH:]


def _t1(aggm, aggu, cntm, cntu, xm, xu, bl1_um, Wr1_um, bl1_mu, Wr1_mu,
        Wl2_um, Wl2_mu):
    return pl.pallas_call(
        _t1_body,
        grid=(GRID,),
        in_specs=[
            _split_spec(), _split_spec(),
            _row_spec(CW), _row_spec(CW),
            _row_spec(H), _row_spec(H),
            _full_spec((1, H)), _full_spec((H, H)),
            _full_spec((1, H)), _full_spec((H, H)),
            _full_spec((H, H)), _full_spec((H, H)),
        ],
        out_specs=[_row_spec(H), _row_spec(H), _split_spec(), _split_spec()],
        out_shape=[
            jax.ShapeDtypeStruct((N, H), jnp.float32),
            jax.ShapeDtypeStruct((N, H), jnp.float32),
            jax.ShapeDtypeStruct((2, N, HH), jnp.float32),
            jax.ShapeDtypeStruct((2, N, HH), jnp.float32),
        ],
    )(aggm, aggu, cntm, cntu, xm, xu, bl1_um.reshape(1, H), Wr1_um,
      bl1_mu.reshape(1, H), Wr1_mu, Wl2_um, Wl2_mu)


def _t2_body(aggm_ref, aggu_ref, cntm_ref, cntu_ref, xm1_ref, xu1_ref,
             bl2um_ref, Wr2um_ref, bl2mu_ref, Wr2mu_ref,
             xm2_ref, xu2_ref):
    rm = 1.0 / jnp.maximum(cntm_ref[...][:, :1], 1.0)
    ru = 1.0 / jnp.maximum(cntu_ref[...][:, :1], 1.0)
    aggm = jnp.concatenate([aggm_ref[0], aggm_ref[1]], axis=-1) * rm
    aggu = jnp.concatenate([aggu_ref[0], aggu_ref[1]], axis=-1) * ru
    xm2_ref[...] = (
        aggm + bl2um_ref[...]
        + jnp.dot(xm1_ref[...], Wr2um_ref[...], preferred_element_type=jnp.float32))
    xu2_ref[...] = (
        aggu + bl2mu_ref[...]
        + jnp.dot(xu1_ref[...], Wr2mu_ref[...], preferred_element_type=jnp.float32))


def _t2(aggm, aggu, cntm, cntu, xm1, xu1, bl2_um, Wr2_um, bl2_mu, Wr2_mu):
    return pl.pallas_call(
        _t2_body,
        grid=(GRID,),
        in_specs=[
            _split_spec(), _split_spec(),
            _row_spec(CW), _row_spec(CW),
            _row_spec(H), _row_spec(H),
            _full_spec((1, H)), _full_spec((H, H)),
            _full_spec((1, H)), _full_spec((H, H)),
        ],
        out_specs=[_row_spec(H), _row_spec(H)],
        out_shape=[
            jax.ShapeDtypeStruct((N, H), jnp.float32),
            jax.ShapeDtypeStruct((N, H), jnp.float32),
        ],
    )(aggm, aggu, cntm, cntu, xm1, xu1, bl2_um.reshape(1, H), Wr2_um,
      bl2_mu.reshape(1, H), Wr2_mu)


def kernel(user_node_id, movie_node_id, movie_x, edge_index, edge_label_index,
           user_emb, movie_emb, lin_W, lin_b,
           Wl1_um, bl1_um, Wr1_um, Wl1_mu, bl1_mu, Wr1_mu,
           Wl2_um, bl2_um, Wr2_um, Wl2_mu, bl2_mu, Wr2_mu):
    xu = user_emb
    src = edge_index[0]
    dst = edge_index[1]

    pad_e = E_PAD - E
    g_src = jnp.concatenate([src, jnp.zeros((pad_e,), jnp.int32)]).reshape(16, CPT, CHUNK)
    s_src = jnp.concatenate([src, jnp.full((pad_e,), TRASH, jnp.int32)]).reshape(16, CPT, CHUNK)
    g_dst = jnp.concatenate([dst, jnp.zeros((pad_e,), jnp.int32)]).reshape(16, CPT, CHUNK)
    s_dst = jnp.concatenate([dst, jnp.full((pad_e,), TRASH, jnp.int32)]).reshape(16, CPT, CHUNK)
    h_src = s_src.reshape(16, ECH, 128)
    h_dst = s_dst.reshape(16, ECH, 128)

    pad_l = EL_PAD - EL
    g_l0 = jnp.concatenate([edge_label_index[0], jnp.zeros((pad_l,), jnp.int32)]).reshape(32, DPT, DCH)
    g_l1 = jnp.concatenate([edge_label_index[1], jnp.zeros((pad_l,), jnp.int32)]).reshape(32, DPT, DCH)

    xm, yu1, ym1 = _t0(movie_x, movie_emb, xu, lin_W, lin_b, Wl1_um, Wl1_mu)

    agg = _make_agg()
    cnt = _make_hist()(h_src, h_dst)
    cntu, cntm = cnt[0], cnt[1]

    aggm1 = agg(yu1, g_src, s_dst)
    aggu1 = agg(ym1, g_dst, s_src)

    xm1, xu1, yu2, ym2 = _t1(aggm1, aggu1, cntm, cntu, xm, xu,
                             bl1_um, Wr1_um, bl1_mu, Wr1_mu, Wl2_um, Wl2_mu)

    aggm2 = agg(yu2, g_src, s_dst)
    aggu2 = agg(ym2, g_dst, s_src)

    xm2, xu2 = _t2(aggm2, aggu2, cntm, cntu, xm1, xu1,
                   bl2_um, Wr2_um, bl2_mu, Wr2_mu)

    pred = _make_decoder()(xu2, xm2, g_l0, g_l1)
    return pred.reshape(EL_PAD)[:EL]
